# baseline (speedup 1.0000x reference)
import functools

import jax
import jax.numpy as jnp
from jax import lax
from jax.experimental import pallas as pl
from jax.experimental.pallas import tpu as pltpu
from jax.experimental.pallas import tpu_sc as plsc

N = 10000
NP = 10240
D_IN = 128
D_HID = 16
D_OUT = 128
E_REAL = 320000
CHUNK = 128
NCHUNK = 80
PER_TILE = CHUNK * NCHUNK
E_PAD = 32 * PER_TILE
ROWS_PER_TILE = NP // 16

_mesh = plsc.VectorSubcoreMesh(core_axis_name="c", subcore_axis_name="s")


def _prop_body(do_gather, feat_hbm, src_hbm, dst_hbm, out_hbm,
               srcv, dstv, rowsv, tmpv, acc, gsem, ssem_a, ssem_b):
  c = lax.axis_index("c")
  s = lax.axis_index("s")
  wid = s * 2 + c
  r0 = s * ROWS_PER_TILE
  if do_gather:
    pltpu.sync_copy(src_hbm.at[wid], srcv)
  else:
    for b in range(16):
      pltpu.sync_copy(feat_hbm.at[pl.ds(0, CHUNK)], rowsv.at[b])
  pltpu.sync_copy(dst_hbm.at[wid], dstv)
  pltpu.sync_copy(feat_hbm.at[pl.ds(r0, ROWS_PER_TILE)], tmpv)
  pltpu.sync_copy(tmpv, acc.at[pl.ds(r0, ROWS_PER_TILE)])
  plsc.subcore_barrier()

  ssems = (ssem_a, ssem_b)
  NG = NCHUNK // 8

  def _drain(b, sem):
    pltpu.make_async_copy(feat_hbm.at[pl.ds(0, CHUNK)], rowsv.at[b],
                          sem).wait()

  if do_gather:
    for b in range(8):
      pltpu.async_copy(feat_hbm.at[srcv.at[b]], rowsv.at[b], gsem)

  def pair(g2, carry):
    for p in range(2):
      g = g2 * 2 + p
      po = 8 * p
      qo = 8 * (1 - p)
      if do_gather:
        for b in range(8):
          _drain(po + b, gsem)

      @pl.when(g > 0)
      def _():
        for b in range(8):
          _drain(qo + b, ssems[1 - p])

      if do_gather:
        @pl.when(g + 1 < NG)
        def _():
          for b in range(8):
            pltpu.async_copy(feat_hbm.at[srcv.at[(g + 1) * 8 + b]],
                             rowsv.at[qo + b], gsem)

      for b in range(8):
        pltpu.async_copy(rowsv.at[po + b], acc.at[dstv.at[g * 8 + b]],
                         ssems[p], add=True)
    return carry

  lax.fori_loop(0, NG // 2, pair, 0)
  last = (NG - 1) % 2
  for b in range(8):
    _drain(8 * last + b, ssems[last])
  plsc.subcore_barrier()
  pltpu.sync_copy(acc.at[pl.ds(r0, ROWS_PER_TILE)], tmpv)
  pltpu.sync_copy(tmpv, out_hbm.at[c, pl.ds(r0, ROWS_PER_TILE), :])


def _make_prop(do_gather):
  return functools.partial(
      pl.kernel,
      out_type=jax.ShapeDtypeStruct((2, NP, D_HID), jnp.float32),
      mesh=_mesh,
      scratch_types=[
          pltpu.VMEM((NCHUNK, CHUNK), jnp.int32),
          pltpu.VMEM((NCHUNK, CHUNK), jnp.int32),
          pltpu.VMEM((16, CHUNK, D_HID), jnp.float32),
          pltpu.VMEM((ROWS_PER_TILE, D_HID), jnp.float32),
          pltpu.VMEM_SHARED((NP, D_HID), jnp.float32),
          pltpu.SemaphoreType.DMA,
          pltpu.SemaphoreType.DMA,
          pltpu.SemaphoreType.DMA,
      ],
      compiler_params=pltpu.CompilerParams(use_tc_tiling_on_sc=False),
  )(functools.partial(_prop_body, do_gather))


_sc_prop = _make_prop(True)
_sc_deg = _make_prop(False)


def _tc_stage1(x_ref, w1_ref, deg_ref, featp_ref, dinv_ref):
  deg = deg_ref[0] + deg_ref[1] - 1.0
  dinv = lax.rsqrt(deg)
  dinv_ref[...] = dinv
  featp_ref[...] = jnp.dot(x_ref[...], w1_ref[...],
                           preferred_element_type=jnp.float32) * dinv


def _tc_mid(m_ref, featp_ref, dinv_ref, b1_ref, out_ref):
  dinv = dinv_ref[...]
  m = m_ref[0] + m_ref[1] - featp_ref[...]
  h = jnp.maximum(m * dinv + b1_ref[...], 0.0)
  out_ref[...] = h * dinv


def _tc_final(m_ref, featp_ref, dinv_ref, w2_ref, b2_ref, out_ref):
  dinv = dinv_ref[...]
  m = (m_ref[0] + m_ref[1] - featp_ref[...]) * dinv
  out_ref[...] = jnp.dot(m, w2_ref[...],
                         preferred_element_type=jnp.float32) + b2_ref[...]


def kernel(V, E, X, W1, b1, W2, b2):
  del V
  f32 = jnp.float32
  src = E[0]
  dst = E[1]
  pad = E_PAD - E_REAL
  src3 = jnp.concatenate([src, jnp.zeros((pad,), jnp.int32)]).reshape(
      32, NCHUNK, CHUNK)
  dst3 = jnp.concatenate([dst, jnp.full((pad,), NP - 1, jnp.int32)]).reshape(
      32, NCHUNK, CHUNK)
  ones = jnp.ones((NP, D_HID), f32)
  Xp = jnp.concatenate([X, jnp.zeros((NP - N, D_IN), f32)])

  degp = _sc_deg(ones, src3, dst3)

  featp, dinv = pl.pallas_call(
      _tc_stage1,
      out_shape=(jax.ShapeDtypeStruct((NP, D_HID), f32),
                 jax.ShapeDtypeStruct((NP, D_HID), f32)),
  )(Xp, W1, degp)

  m1 = _sc_prop(featp, src3, dst3)

  hp = pl.pallas_call(
      _tc_mid,
      out_shape=jax.ShapeDtypeStruct((NP, D_HID), f32),
  )(m1, featp, dinv, b1.reshape(1, D_HID))

  m2 = _sc_prop(hp, src3, dst3)

  out = pl.pallas_call(
      _tc_final,
      out_shape=jax.ShapeDtypeStruct((NP, D_OUT), f32),
  )(m2, hp, dinv, W2, b2.reshape(1, D_OUT))
  return out[:N]


# speedup vs baseline: 1.4272x; 1.4272x over previous
import functools

import jax
import jax.numpy as jnp
from jax import lax
from jax.experimental import pallas as pl
from jax.experimental.pallas import tpu as pltpu
from jax.experimental.pallas import tpu_sc as plsc

N = 10000
NP = 10240
D_IN = 128
D_HID = 16
D_OUT = 128
E_REAL = 320000
CHUNK = 128
NCHUNK = 80
PER_TILE = CHUNK * NCHUNK
E_PAD = 32 * PER_TILE
ROWS_PER_TILE = NP // 16

_mesh = plsc.VectorSubcoreMesh(core_axis_name="c", subcore_axis_name="s")


def _prop_body(do_gather, feat_hbm, src_hbm, dst_hbm, out_hbm,
               srcv, dstv, rowsv, tmpv, acc, featsh, gsem, ssem_a, ssem_b):
  c = lax.axis_index("c")
  s = lax.axis_index("s")
  wid = s * 2 + c
  r0 = s * ROWS_PER_TILE
  if do_gather:
    pltpu.sync_copy(src_hbm.at[wid], srcv)
  else:
    for b in range(16):
      pltpu.sync_copy(feat_hbm.at[pl.ds(0, CHUNK)], rowsv.at[b])
  pltpu.sync_copy(dst_hbm.at[wid], dstv)
  pltpu.sync_copy(feat_hbm.at[pl.ds(r0, ROWS_PER_TILE)], tmpv)
  pltpu.sync_copy(tmpv, acc.at[pl.ds(r0, ROWS_PER_TILE)])
  if do_gather:
    pltpu.sync_copy(tmpv, featsh.at[pl.ds(r0, ROWS_PER_TILE)])
  plsc.subcore_barrier()

  ssems = (ssem_a, ssem_b)
  NG = NCHUNK // 8

  def _drain(b, sem):
    pltpu.make_async_copy(feat_hbm.at[pl.ds(0, CHUNK)], rowsv.at[b],
                          sem).wait()

  if do_gather:
    for b in range(8):
      pltpu.async_copy(featsh.at[srcv.at[b]], rowsv.at[b], gsem)

  def pair(g2, carry):
    for p in range(2):
      g = g2 * 2 + p
      po = 8 * p
      qo = 8 * (1 - p)
      if do_gather:
        for b in range(8):
          _drain(po + b, gsem)

      @pl.when(g > 0)
      def _():
        for b in range(8):
          _drain(qo + b, ssems[1 - p])

      if do_gather:
        @pl.when(g + 1 < NG)
        def _():
          for b in range(8):
            pltpu.async_copy(featsh.at[srcv.at[(g + 1) * 8 + b]],
                             rowsv.at[qo + b], gsem)

      for b in range(8):
        pltpu.async_copy(rowsv.at[po + b], acc.at[dstv.at[g * 8 + b]],
                         ssems[p], add=True)
    return carry

  lax.fori_loop(0, NG // 2, pair, 0)
  last = (NG - 1) % 2
  for b in range(8):
    _drain(8 * last + b, ssems[last])
  plsc.subcore_barrier()
  pltpu.sync_copy(acc.at[pl.ds(r0, ROWS_PER_TILE)], tmpv)
  pltpu.sync_copy(tmpv, out_hbm.at[c, pl.ds(r0, ROWS_PER_TILE), :])


def _make_prop(do_gather):
  return functools.partial(
      pl.kernel,
      out_type=jax.ShapeDtypeStruct((2, NP, D_HID), jnp.float32),
      mesh=_mesh,
      scratch_types=[
          pltpu.VMEM((NCHUNK, CHUNK), jnp.int32),
          pltpu.VMEM((NCHUNK, CHUNK), jnp.int32),
          pltpu.VMEM((16, CHUNK, D_HID), jnp.float32),
          pltpu.VMEM((ROWS_PER_TILE, D_HID), jnp.float32),
          pltpu.VMEM_SHARED((NP, D_HID), jnp.float32),
          pltpu.VMEM_SHARED((NP, D_HID), jnp.float32),
          pltpu.SemaphoreType.DMA,
          pltpu.SemaphoreType.DMA,
          pltpu.SemaphoreType.DMA,
      ],
      compiler_params=pltpu.CompilerParams(use_tc_tiling_on_sc=False),
  )(functools.partial(_prop_body, do_gather))


_sc_prop = _make_prop(True)
_sc_deg = _make_prop(False)


def _tc_stage1(x_ref, w1_ref, deg_ref, featp_ref, dinv_ref):
  deg = deg_ref[0] + deg_ref[1] - 1.0
  dinv = lax.rsqrt(deg)
  dinv_ref[...] = dinv
  featp_ref[...] = jnp.dot(x_ref[...], w1_ref[...],
                           preferred_element_type=jnp.float32) * dinv


def _tc_mid(m_ref, featp_ref, dinv_ref, b1_ref, out_ref):
  dinv = dinv_ref[...]
  m = m_ref[0] + m_ref[1] - featp_ref[...]
  h = jnp.maximum(m * dinv + b1_ref[...], 0.0)
  out_ref[...] = h * dinv


def _tc_final(m_ref, featp_ref, dinv_ref, w2_ref, b2_ref, out_ref):
  dinv = dinv_ref[...]
  m = (m_ref[0] + m_ref[1] - featp_ref[...]) * dinv
  out_ref[...] = jnp.dot(m, w2_ref[...],
                         preferred_element_type=jnp.float32) + b2_ref[...]


def kernel(V, E, X, W1, b1, W2, b2):
  del V
  f32 = jnp.float32
  src = E[0]
  dst = E[1]
  pad = E_PAD - E_REAL
  src3 = jnp.concatenate([src, jnp.zeros((pad,), jnp.int32)]).reshape(
      32, NCHUNK, CHUNK)
  dst3 = jnp.concatenate([dst, jnp.full((pad,), NP - 1, jnp.int32)]).reshape(
      32, NCHUNK, CHUNK)
  ones = jnp.ones((NP, D_HID), f32)
  Xp = jnp.concatenate([X, jnp.zeros((NP - N, D_IN), f32)])

  degp = _sc_deg(ones, src3, dst3)

  featp, dinv = pl.pallas_call(
      _tc_stage1,
      out_shape=(jax.ShapeDtypeStruct((NP, D_HID), f32),
                 jax.ShapeDtypeStruct((NP, D_HID), f32)),
  )(Xp, W1, degp)

  m1 = _sc_prop(featp, src3, dst3)

  hp = pl.pallas_call(
      _tc_mid,
      out_shape=jax.ShapeDtypeStruct((NP, D_HID), f32),
  )(m1, featp, dinv, b1.reshape(1, D_HID))

  m2 = _sc_prop(hp, src3, dst3)

  out = pl.pallas_call(
      _tc_final,
      out_shape=jax.ShapeDtypeStruct((NP, D_OUT), f32),
  )(m2, hp, dinv, W2, b2.reshape(1, D_OUT))
  return out[:N]


# speedup vs baseline: 1.4463x; 1.0134x over previous
import functools

import jax
import jax.numpy as jnp
from jax import lax
from jax.experimental import pallas as pl
from jax.experimental.pallas import tpu as pltpu
from jax.experimental.pallas import tpu_sc as plsc

N = 10000
NP = 10240
D_IN = 128
D_HID = 16
D_OUT = 128
E_REAL = 320000
CHUNK = 128
NCHUNK = 80
PER_TILE = CHUNK * NCHUNK
E_PAD = 32 * PER_TILE
ROWS_PER_TILE = NP // 16

_mesh = plsc.VectorSubcoreMesh(core_axis_name="c", subcore_axis_name="s")


def _prop_body(do_gather, feat_hbm, src_hbm, dst_hbm, out_hbm,
               srcv, dstv, rowsv, tmpv, acc, featsh, gsem, ssem_a, ssem_b):
  c = lax.axis_index("c")
  s = lax.axis_index("s")
  wid = s * 2 + c
  r0 = s * ROWS_PER_TILE
  if do_gather:
    pltpu.sync_copy(src_hbm.at[wid], srcv)
  else:
    for b in range(16):
      pltpu.sync_copy(feat_hbm.at[pl.ds(0, CHUNK)], rowsv.at[b])
  pltpu.sync_copy(dst_hbm.at[wid], dstv)
  pltpu.sync_copy(feat_hbm.at[pl.ds(r0, ROWS_PER_TILE)], tmpv)
  pltpu.sync_copy(tmpv, acc.at[pl.ds(r0, ROWS_PER_TILE)])
  if do_gather:
    pltpu.sync_copy(tmpv, featsh.at[pl.ds(r0, ROWS_PER_TILE)])
  plsc.subcore_barrier()

  ssems = (ssem_a, ssem_b)
  NG = NCHUNK // 8

  def _drain(b, sem):
    pltpu.make_async_copy(feat_hbm.at[pl.ds(0, CHUNK)], rowsv.at[b],
                          sem).wait()

  if do_gather:
    for b in range(8):
      pltpu.async_copy(featsh.at[srcv.at[b]], rowsv.at[b], gsem)

  def pair(g2, carry):
    for p in range(2):
      g = g2 * 2 + p
      po = 8 * p
      qo = 8 * (1 - p)
      if do_gather:
        for b in range(8):
          _drain(po + b, gsem)

      @pl.when(g > 0)
      def _():
        for b in range(8):
          _drain(qo + b, ssems[1 - p])

      if do_gather:
        @pl.when(g + 1 < NG)
        def _():
          for b in range(8):
            pltpu.async_copy(featsh.at[srcv.at[(g + 1) * 8 + b]],
                             rowsv.at[qo + b], gsem)

      for b in range(8):
        pltpu.async_copy(rowsv.at[po + b], acc.at[dstv.at[g * 8 + b]],
                         ssems[p], add=True)
    return carry

  lax.fori_loop(0, NG // 2, pair, 0)
  last = (NG - 1) % 2
  for b in range(8):
    _drain(8 * last + b, ssems[last])
  plsc.subcore_barrier()
  pltpu.sync_copy(acc.at[pl.ds(r0, ROWS_PER_TILE)], tmpv)
  pltpu.sync_copy(tmpv, out_hbm.at[c, pl.ds(r0, ROWS_PER_TILE), :])


def _make_prop(do_gather, width):
  return functools.partial(
      pl.kernel,
      out_type=jax.ShapeDtypeStruct((2, NP, width), jnp.float32),
      mesh=_mesh,
      scratch_types=[
          pltpu.VMEM((NCHUNK, CHUNK), jnp.int32),
          pltpu.VMEM((NCHUNK, CHUNK), jnp.int32),
          pltpu.VMEM((16, CHUNK, width), jnp.float32),
          pltpu.VMEM((ROWS_PER_TILE, width), jnp.float32),
          pltpu.VMEM_SHARED((NP, width), jnp.float32),
          pltpu.VMEM_SHARED((NP, width), jnp.float32),
          pltpu.SemaphoreType.DMA,
          pltpu.SemaphoreType.DMA,
          pltpu.SemaphoreType.DMA,
      ],
      compiler_params=pltpu.CompilerParams(use_tc_tiling_on_sc=False),
  )(functools.partial(_prop_body, do_gather))


_sc_prop = _make_prop(True, D_HID)
_sc_deg = _make_prop(False, 8)


def _tc_stage1(x_ref, w1_ref, deg_ref, featp_ref, dinv_ref):
  deg = (deg_ref[0] + deg_ref[1])[:, :1] - 1.0
  dinv = lax.rsqrt(deg)
  dinv_ref[...] = dinv
  featp_ref[...] = jnp.dot(x_ref[...], w1_ref[...],
                           preferred_element_type=jnp.float32) * dinv


def _tc_mid(m_ref, featp_ref, dinv_ref, b1_ref, out_ref):
  dinv = dinv_ref[...]
  m = m_ref[0] + m_ref[1] - featp_ref[...]
  h = jnp.maximum(m * dinv + b1_ref[...], 0.0)
  out_ref[...] = h * dinv


def _tc_final(m_ref, featp_ref, dinv_ref, w2_ref, b2_ref, out_ref):
  dinv = dinv_ref[...]
  m = (m_ref[0] + m_ref[1] - featp_ref[...]) * dinv
  out_ref[...] = jnp.dot(m, w2_ref[...],
                         preferred_element_type=jnp.float32) + b2_ref[...]


def kernel(V, E, X, W1, b1, W2, b2):
  del V
  f32 = jnp.float32
  src = E[0]
  dst = E[1]
  pad = E_PAD - E_REAL
  src3 = jnp.concatenate([src, jnp.zeros((pad,), jnp.int32)]).reshape(
      32, NCHUNK, CHUNK)
  dst3 = jnp.concatenate([dst, jnp.full((pad,), NP - 1, jnp.int32)]).reshape(
      32, NCHUNK, CHUNK)
  ones = jnp.ones((NP, 8), f32)
  Xp = jnp.concatenate([X, jnp.zeros((NP - N, D_IN), f32)])

  degp = _sc_deg(ones, src3, dst3)

  featp, dinv = pl.pallas_call(
      _tc_stage1,
      out_shape=(jax.ShapeDtypeStruct((NP, D_HID), f32),
                 jax.ShapeDtypeStruct((NP, 1), f32)),
  )(Xp, W1, degp)

  m1 = _sc_prop(featp, src3, dst3)

  hp = pl.pallas_call(
      _tc_mid,
      out_shape=jax.ShapeDtypeStruct((NP, D_HID), f32),
  )(m1, featp, dinv, b1.reshape(1, D_HID))

  m2 = _sc_prop(hp, src3, dst3)

  out = pl.pallas_call(
      _tc_final,
      out_shape=jax.ShapeDtypeStruct((NP, D_OUT), f32),
  )(m2, hp, dinv, W2, b2.reshape(1, D_OUT))
  return out[:N]


# speedup vs baseline: 1.5649x; 1.0820x over previous
import functools

import jax
import jax.numpy as jnp
from jax import lax
from jax.experimental import pallas as pl
from jax.experimental.pallas import tpu as pltpu
from jax.experimental.pallas import tpu_sc as plsc

N = 10000
NP = 10240
D_IN = 128
D_HID = 16
D_OUT = 128
E_REAL = 320000
CHUNK = 128
NCHUNK = 80
PER_TILE = E_REAL // 32
FULL_CHUNKS = PER_TILE // CHUNK
REM = PER_TILE - FULL_CHUNKS * CHUNK
ROWS_PER_TILE = NP // 16

_mesh = plsc.VectorSubcoreMesh(core_axis_name="c", subcore_axis_name="s")


def _load_edges(e_hbm, row, buf, base, sem, pad_val):
  handles = [
      pltpu.async_copy(e_hbm.at[row, pl.ds(base + j * CHUNK, CHUNK)],
                       buf.at[j], sem)
      for j in range(FULL_CHUNKS)
  ]
  handles.append(
      pltpu.async_copy(e_hbm.at[row, pl.ds(base + FULL_CHUNKS * CHUNK, REM)],
                       buf.at[FULL_CHUNKS, pl.ds(0, REM)], sem))
  pad = jnp.full((16,), pad_val, jnp.int32)
  for k in range(REM // 16, CHUNK // 16):
    buf[FULL_CHUNKS, pl.ds(16 * k, 16)] = pad
  for k in range(CHUNK // 16):
    buf[NCHUNK - 1, pl.ds(16 * k, 16)] = pad
  return handles


def _prop_body(do_gather, feat_hbm, e_hbm, out_hbm,
               srcv, dstv, rowsv, tmpv, acc, featsh, gsem, ssem_a, ssem_b):
  c = lax.axis_index("c")
  s = lax.axis_index("s")
  wid = s * 2 + c
  r0 = s * ROWS_PER_TILE
  ebase = wid * PER_TILE
  handles = _load_edges(e_hbm, 1, dstv, ebase, gsem, NP - 1)
  if do_gather:
    handles += _load_edges(e_hbm, 0, srcv, ebase, gsem, 0)
  else:
    for b in range(16):
      pltpu.sync_copy(feat_hbm.at[pl.ds(0, CHUNK)], rowsv.at[b])
  pltpu.sync_copy(feat_hbm.at[pl.ds(r0, ROWS_PER_TILE)], tmpv)
  pltpu.sync_copy(tmpv, acc.at[pl.ds(r0, ROWS_PER_TILE)])
  if do_gather:
    pltpu.sync_copy(tmpv, featsh.at[pl.ds(r0, ROWS_PER_TILE)])
  for h in handles:
    h.wait()
  plsc.subcore_barrier()

  ssems = (ssem_a, ssem_b)
  NG = NCHUNK // 8

  def _drain(b, sem):
    pltpu.make_async_copy(feat_hbm.at[pl.ds(0, CHUNK)], rowsv.at[b],
                          sem).wait()

  if do_gather:
    for b in range(8):
      pltpu.async_copy(featsh.at[srcv.at[b]], rowsv.at[b], gsem)

  def pair(g2, carry):
    for p in range(2):
      g = g2 * 2 + p
      po = 8 * p
      qo = 8 * (1 - p)
      if do_gather:
        for b in range(8):
          _drain(po + b, gsem)

      @pl.when(g > 0)
      def _():
        for b in range(8):
          _drain(qo + b, ssems[1 - p])

      if do_gather:
        @pl.when(g + 1 < NG)
        def _():
          for b in range(8):
            pltpu.async_copy(featsh.at[srcv.at[(g + 1) * 8 + b]],
                             rowsv.at[qo + b], gsem)

      for b in range(8):
        pltpu.async_copy(rowsv.at[po + b], acc.at[dstv.at[g * 8 + b]],
                         ssems[p], add=True)
    return carry

  lax.fori_loop(0, NG // 2, pair, 0)
  last = (NG - 1) % 2
  for b in range(8):
    _drain(8 * last + b, ssems[last])
  plsc.subcore_barrier()
  pltpu.sync_copy(acc.at[pl.ds(r0, ROWS_PER_TILE)], tmpv)
  pltpu.sync_copy(tmpv, out_hbm.at[c, pl.ds(r0, ROWS_PER_TILE), :])


def _make_prop(do_gather, width):
  return functools.partial(
      pl.kernel,
      out_type=jax.ShapeDtypeStruct((2, NP, width), jnp.float32),
      mesh=_mesh,
      scratch_types=[
          pltpu.VMEM((NCHUNK, CHUNK), jnp.int32),
          pltpu.VMEM((NCHUNK, CHUNK), jnp.int32),
          pltpu.VMEM((16, CHUNK, width), jnp.float32),
          pltpu.VMEM((ROWS_PER_TILE, width), jnp.float32),
          pltpu.VMEM_SHARED((NP, width), jnp.float32),
          pltpu.VMEM_SHARED((NP, width), jnp.float32),
          pltpu.SemaphoreType.DMA,
          pltpu.SemaphoreType.DMA,
          pltpu.SemaphoreType.DMA,
      ],
      compiler_params=pltpu.CompilerParams(use_tc_tiling_on_sc=False),
  )(functools.partial(_prop_body, do_gather))


_sc_prop = _make_prop(True, D_HID)
_sc_deg = _make_prop(False, 8)


def _tc_stage1(x_ref, w1_ref, deg_ref, featp_ref, dinv_ref):
  deg = (deg_ref[0] + deg_ref[1])[:, :1] - 1.0
  dinv = lax.rsqrt(deg)
  dinv_ref[...] = dinv
  featp_ref[...] = jnp.dot(x_ref[...], w1_ref[...],
                           preferred_element_type=jnp.float32) * dinv


def _tc_mid(m_ref, featp_ref, dinv_ref, b1_ref, out_ref):
  dinv = dinv_ref[...]
  m = m_ref[0] + m_ref[1] - featp_ref[...]
  h = jnp.maximum(m * dinv + b1_ref[...], 0.0)
  out_ref[...] = h * dinv


def _tc_final(m_ref, featp_ref, dinv_ref, w2_ref, b2_ref, out_ref):
  dinv = dinv_ref[...]
  m = (m_ref[0] + m_ref[1] - featp_ref[...]) * dinv
  out_ref[...] = jnp.dot(m, w2_ref[...],
                         preferred_element_type=jnp.float32) + b2_ref[...]


def kernel(V, E, X, W1, b1, W2, b2):
  del V
  f32 = jnp.float32
  ones = jnp.ones((NP, 8), f32)
  Xp = jnp.concatenate([X, jnp.zeros((NP - N, D_IN), f32)])

  degp = _sc_deg(ones, E)

  featp, dinv = pl.pallas_call(
      _tc_stage1,
      out_shape=(jax.ShapeDtypeStruct((NP, D_HID), f32),
                 jax.ShapeDtypeStruct((NP, 1), f32)),
  )(Xp, W1, degp)

  m1 = _sc_prop(featp, E)

  hp = pl.pallas_call(
      _tc_mid,
      out_shape=jax.ShapeDtypeStruct((NP, D_HID), f32),
  )(m1, featp, dinv, b1.reshape(1, D_HID))

  m2 = _sc_prop(hp, E)

  out = pl.pallas_call(
      _tc_final,
      out_shape=jax.ShapeDtypeStruct((NP, D_OUT), f32),
  )(m2, hp, dinv, W2, b2.reshape(1, D_OUT))
  return out[:N]
